# baseline (device time: 222677 ns/iter reference)
import jax
import jax.numpy as jnp
from jax import lax
from jax.experimental import pallas as pl
from jax.experimental.pallas import tpu as pltpu

N_DEV = 4
CH = 16
CORR_T = 64


def kernel(x, A, B, C):
    Bb, S, D = x.shape
    N = A.shape[1]
    n_chunks = S // CH

    dAT = jnp.exp(A).T
    B4 = (B.transpose(2, 0, 1).reshape(N, Bb, n_chunks, CH)
          .transpose(2, 0, 1, 3).astype(jnp.bfloat16))
    C4 = (C.transpose(2, 0, 1).reshape(N, Bb, n_chunks, CH)
          .transpose(2, 0, 1, 3).astype(jnp.bfloat16))

    def body(x_ref, dAT_ref, B_ref, C_ref, out_ref,
             x16_ref, send_ref, recv_ref, send_sem, recv_sem):
        my = lax.axis_index("i")
        left = (my - 1) % N_DEV
        right = (my + 1) % N_DEV

        dAT_f32 = dAT_ref[...]
        dAT_v = dAT_f32.astype(jnp.bfloat16)

        def cvt(c, _):
            sl = pl.ds(c * CH, CH)
            x16_ref[:, sl, :] = x_ref[:, sl, :].astype(jnp.bfloat16)
            return 0
        lax.fori_loop(0, n_chunks, cvt, 0)

        def chunk(c, h):
            t0 = c * CH
            xc = x16_ref[:, pl.ds(t0, CH), :]
            Bc = B_ref[c]
            Cc = C_ref[c]
            ys = []
            for j in range(CH):
                xj = xc[:, j, :][None, :, :]
                bj = Bc[:, :, j][:, :, None]
                cj = Cc[:, :, j][:, :, None]
                h = h * dAT_v[:, None, :] + xj * bj
                ys.append(jnp.sum(h * cj, axis=0))
            out_ref[:, pl.ds(t0, CH), :] = (
                jnp.stack(ys, axis=1).astype(jnp.float32))
            return h

        h0 = jnp.zeros((N, Bb, D), jnp.bfloat16)
        h_final = lax.fori_loop(0, n_chunks, chunk, h0)

        shift = pltpu.make_async_remote_copy(
            src_ref=send_ref, dst_ref=recv_ref,
            send_sem=send_sem, recv_sem=recv_sem,
            device_id=(right,), device_id_type=pl.DeviceIdType.MESH,
        )

        @pl.when(my < N_DEV - 1)
        def _():
            send_ref[...] = h_final
            shift.start()
            shift.wait_send()

        @pl.when(my > 0)
        def _():
            shift.wait_recv()

        carry = jnp.where(
            my == 0, 0.0, recv_ref[...].astype(jnp.float32))

        def corr(c, g):
            t0 = c * CH
            Cc = C_ref[c]
            ys = []
            for j in range(CH):
                g = g * dAT_f32[:, None, :]
                cj = Cc[:, :, j][:, :, None].astype(jnp.float32)
                ys.append(jnp.sum(g * cj, axis=0))
            out_ref[:, pl.ds(t0, CH), :] += jnp.stack(ys, axis=1)
            return g

        lax.fori_loop(0, CORR_T // CH, corr, carry)

    return pl.pallas_call(
        body,
        out_shape=jax.ShapeDtypeStruct((Bb, S, D), jnp.float32),
        in_specs=[pl.BlockSpec(memory_space=pltpu.VMEM)] * 4,
        out_specs=pl.BlockSpec(memory_space=pltpu.VMEM),
        scratch_shapes=[
            pltpu.VMEM((Bb, S, D), jnp.bfloat16),
            pltpu.VMEM((N, Bb, D), jnp.bfloat16),
            pltpu.VMEM((N, Bb, D), jnp.bfloat16),
            pltpu.SemaphoreType.DMA,
            pltpu.SemaphoreType.DMA,
        ],
    )(x, dAT, B4, C4)


# device time: 61779 ns/iter; 3.6044x vs baseline; 3.6044x over previous
import jax
import jax.numpy as jnp
from jax import lax
from jax.experimental import pallas as pl
from jax.experimental.pallas import tpu as pltpu

N_DEV = 4
CH = 32
CORR_T = 64


def kernel(x, A, B, C):
    Bb, S, D = x.shape
    N = A.shape[1]
    n_chunks = S // CH

    dAT = jnp.exp(A).T
    B16 = B.astype(jnp.bfloat16)
    C16 = C.astype(jnp.bfloat16)

    def body(x_ref, dAT_ref, B_ref, C_ref, out_ref,
             x16_ref, send_ref, recv_ref, send_sem, recv_sem):
        my = lax.axis_index("i")
        left = (my - 1) % N_DEV
        right = (my + 1) % N_DEV

        dAT_f32 = dAT_ref[...]
        dAT_v = dAT_f32.astype(jnp.bfloat16)

        def cvt(c, _):
            sl = pl.ds(c * CH, CH)
            x16_ref[:, sl, :] = x_ref[:, sl, :].astype(jnp.bfloat16)
            return 0
        lax.fori_loop(0, n_chunks, cvt, 0)

        def chunk(c, h):
            t0 = c * CH
            xc = x16_ref[:, pl.ds(t0, CH), :]
            Bc = B_ref[:, pl.ds(t0, CH), :]
            Cc = C_ref[:, pl.ds(t0, CH), :]
            ys = []
            for j in range(CH):
                xj = xc[:, j, :][:, None, :]
                bj = Bc[:, j, :][:, :, None]
                cj = Cc[:, j, :][:, :, None]
                h = h * dAT_v[None] + xj * bj
                ys.append(jnp.sum(h * cj, axis=1))
            out_ref[:, pl.ds(t0, CH), :] = (
                jnp.stack(ys, axis=1).astype(jnp.float32))
            return h

        h0 = jnp.zeros((Bb, N, D), jnp.bfloat16)
        h_final = lax.fori_loop(0, n_chunks, chunk, h0)

        shift = pltpu.make_async_remote_copy(
            src_ref=send_ref, dst_ref=recv_ref,
            send_sem=send_sem, recv_sem=recv_sem,
            device_id=(right,), device_id_type=pl.DeviceIdType.MESH,
        )

        @pl.when(my < N_DEV - 1)
        def _():
            send_ref[...] = h_final
            shift.start()
            shift.wait_send()

        @pl.when(my > 0)
        def _():
            shift.wait_recv()

        carry = jnp.where(
            my == 0, 0.0, recv_ref[...].astype(jnp.float32))

        def corr(c, g):
            t0 = c * CH
            Cc = C_ref[:, pl.ds(t0, CH), :]
            ys = []
            for j in range(CH):
                g = g * dAT_f32[None]
                cj = Cc[:, j, :][:, :, None].astype(jnp.float32)
                ys.append(jnp.sum(g * cj, axis=1))
            out_ref[:, pl.ds(t0, CH), :] += jnp.stack(ys, axis=1)
            return g

        lax.fori_loop(0, CORR_T // CH, corr, carry)

    return pl.pallas_call(
        body,
        out_shape=jax.ShapeDtypeStruct((Bb, S, D), jnp.float32),
        in_specs=[pl.BlockSpec(memory_space=pltpu.VMEM)] * 4,
        out_specs=pl.BlockSpec(memory_space=pltpu.VMEM),
        scratch_shapes=[
            pltpu.VMEM((Bb, S, D), jnp.bfloat16),
            pltpu.VMEM((Bb, N, D), jnp.bfloat16),
            pltpu.VMEM((Bb, N, D), jnp.bfloat16),
            pltpu.SemaphoreType.DMA,
            pltpu.SemaphoreType.DMA,
        ],
    )(x, dAT, B16, C16)
